# Initial kernel scaffold; baseline (speedup 1.0000x reference)
import jax
import jax.numpy as jnp
from jax.experimental import pallas as pl


def kernel(x, edge_index, edge_attr, adj, W_rel_e, W_root_e, b_root_e, ln_g, ln_b, Wq, bq, Wk, bk, Wv, bv, We, Wskip, bskip, W_rel_d, W_root_d, b_root_d):
    raise NotImplementedError("write your pallas kernel here")


# speedup vs baseline: 1.0028x; 1.0028x over previous
import functools

import jax
import jax.numpy as jnp
from jax.experimental import pallas as pl
from jax.experimental.pallas import tpu as pltpu

N_NODES = 10000
HID = 128
N_LAYERS = 5
BN = 400


def _sage_body(adj_ref, xf_ref, xr_ref, wrel_ref, wroot_ref, b_ref, g_ref,
               bln_ref, o_ref, *, ln: bool):
    a = adj_ref[...]
    agg = jnp.dot(a, xf_ref[...], preferred_element_type=jnp.float32)
    deg = jnp.clip(jnp.sum(a, axis=1, keepdims=True), 1.0, None)
    h = (jnp.dot(agg / deg, wrel_ref[...], preferred_element_type=jnp.float32)
         + jnp.dot(xr_ref[...], wroot_ref[...], preferred_element_type=jnp.float32)
         + b_ref[...])
    if ln:
        mu = jnp.mean(h, axis=1, keepdims=True)
        var = jnp.mean((h - mu) ** 2, axis=1, keepdims=True)
        h = (h - mu) * jax.lax.rsqrt(var + 1e-5) * g_ref[...] + bln_ref[...]
        h = jnp.maximum(h, 0.0)
    o_ref[...] = h


def _dense_sage(adj, xf, w_rel, w_root, b, g, bln, ln):
    n = xf.shape[0]
    fin = xf.shape[1]
    fout = w_rel.shape[1]
    grid = (n // BN,)
    return pl.pallas_call(
        functools.partial(_sage_body, ln=ln),
        grid=grid,
        in_specs=[
            pl.BlockSpec((BN, n), lambda i: (i, 0)),
            pl.BlockSpec((n, fin), lambda i: (0, 0)),
            pl.BlockSpec((BN, fin), lambda i: (i, 0)),
            pl.BlockSpec((fin, fout), lambda i: (0, 0)),
            pl.BlockSpec((fin, fout), lambda i: (0, 0)),
            pl.BlockSpec((1, fout), lambda i: (0, 0)),
            pl.BlockSpec((1, fout), lambda i: (0, 0)),
            pl.BlockSpec((1, fout), lambda i: (0, 0)),
        ],
        out_specs=pl.BlockSpec((BN, fout), lambda i: (i, 0)),
        out_shape=jax.ShapeDtypeStruct((n, fout), jnp.float32),
    )(adj, xf, xf, w_rel, w_root, b.reshape(1, -1), g.reshape(1, -1),
      bln.reshape(1, -1))


def _transformer_conv(h, src, dst, e_attr, Wq, bq, Wk, bk, Wv, bv, We,
                      Wskip, bskip):
    q = h @ Wq + bq
    k = h @ Wk + bk
    v = h @ Wv + bv
    e = e_attr @ We
    kj = k[src] + e
    vj = v[src] + e
    alpha = jnp.sum(q[dst] * kj, axis=-1) / jnp.sqrt(jnp.float32(HID))
    amax = jax.ops.segment_max(alpha, dst, num_segments=N_NODES)
    amax = jnp.where(jnp.isfinite(amax), amax, 0.0)
    ex = jnp.exp(alpha - amax[dst])
    den = jax.ops.segment_sum(ex, dst, num_segments=N_NODES)
    a = ex / (den[dst] + 1e-16)
    out = jax.ops.segment_sum(a[:, None] * vj, dst, num_segments=N_NODES)
    return out + (h @ Wskip + bskip)


def kernel(x, edge_index, edge_attr, adj, W_rel_e, W_root_e, b_root_e, ln_g,
           ln_b, Wq, bq, Wk, bk, Wv, bv, We, Wskip, bskip, W_rel_d, W_root_d,
           b_root_d):
    src = edge_index[0]
    dst = edge_index[1]
    xf = x.reshape(N_NODES, -1)
    adj2 = adj.reshape(N_NODES, N_NODES)
    h = _dense_sage(adj2, xf, W_rel_e, W_root_e, b_root_e, ln_g, ln_b, True)
    for i in range(N_LAYERS):
        h = jax.nn.relu(_transformer_conv(
            h, src, dst, edge_attr, Wq[i], bq[i], Wk[i], bk[i], Wv[i], bv[i],
            We[i], Wskip[i], bskip[i]))
    out = _dense_sage(adj2, h, W_rel_d, W_root_d, b_root_d, ln_g, ln_b, False)
    return out.reshape(N_NODES, HID, 1)


# speedup vs baseline: 1.7445x; 1.7396x over previous
import functools

import jax
import jax.numpy as jnp
from jax import lax
from jax.experimental import pallas as pl
from jax.experimental.pallas import tpu as pltpu
from jax.experimental.pallas import tpu_sc as plsc

N = 10000
E = 320000
HID = 128
EDIM = 16
N_LAYERS = 5
BN = 400
BNP = 2000
NW = 32
EPW = E // NW
C = 80
NCHUNK = EPW // C
NGRP = C // 16
RPT = 632
RPT_LAST = N - 15 * RPT
FEROWS = 1256
INV_SQRT = float(1.0 / __import__("math").sqrt(HID))
HI = jax.lax.Precision.HIGHEST


def _sage_body(adj_ref, xf_ref, xr_ref, wrel_ref, wroot_ref, b_ref, g_ref,
               bln_ref, o_ref, *, ln: bool):
    a = adj_ref[...]
    agg = jnp.dot(a, xf_ref[...], preferred_element_type=jnp.float32)
    deg = jnp.clip(jnp.sum(a, axis=1, keepdims=True), 1.0, None)
    h = (jnp.dot(agg / deg, wrel_ref[...], preferred_element_type=jnp.float32,
                 precision=HI)
         + jnp.dot(xr_ref[...], wroot_ref[...],
                   preferred_element_type=jnp.float32, precision=HI)
         + b_ref[...])
    if ln:
        mu = jnp.mean(h, axis=1, keepdims=True)
        var = jnp.mean((h - mu) ** 2, axis=1, keepdims=True)
        h = (h - mu) * jax.lax.rsqrt(var + 1e-5) * g_ref[...] + bln_ref[...]
        h = jnp.maximum(h, 0.0)
    o_ref[...] = h


def _dense_sage(adj, xf, w_rel, w_root, b, g, bln, ln):
    n, fin = xf.shape
    fout = w_rel.shape[1]
    return pl.pallas_call(
        functools.partial(_sage_body, ln=ln),
        grid=(n // BN,),
        in_specs=[
            pl.BlockSpec((BN, n), lambda i: (i, 0)),
            pl.BlockSpec((n, fin), lambda i: (0, 0)),
            pl.BlockSpec((BN, fin), lambda i: (i, 0)),
            pl.BlockSpec((fin, fout), lambda i: (0, 0)),
            pl.BlockSpec((fin, fout), lambda i: (0, 0)),
            pl.BlockSpec((1, fout), lambda i: (0, 0)),
            pl.BlockSpec((1, fout), lambda i: (0, 0)),
            pl.BlockSpec((1, fout), lambda i: (0, 0)),
        ],
        out_specs=pl.BlockSpec((BN, fout), lambda i: (i, 0)),
        out_shape=jax.ShapeDtypeStruct((n, fout), jnp.float32),
    )(adj, xf, xf, w_rel, w_root, b.reshape(1, -1), g.reshape(1, -1),
      bln.reshape(1, -1))


def _proj_body(h_ref, wq_ref, wk_ref, wv_ref, ws_ref, bq_ref, bk_ref, bv_ref,
               bs_ref, we_ref, qcat_ref, k_ref, v_ref, s_ref):
    hh = h_ref[...]
    q = jnp.dot(hh, wq_ref[...], preferred_element_type=jnp.float32,
                precision=HI) + bq_ref[...]
    qe = lax.dot_general(q, we_ref[...], (((1,), (1,)), ((), ())),
                         preferred_element_type=jnp.float32, precision=HI)
    qcat_ref[...] = jnp.concatenate(
        [q, qe, jnp.zeros((q.shape[0], 2 * HID - HID - EDIM), jnp.float32)],
        axis=1)
    k_ref[...] = jnp.dot(hh, wk_ref[...], preferred_element_type=jnp.float32,
                         precision=HI) + bk_ref[...]
    v_ref[...] = jnp.dot(hh, wv_ref[...], preferred_element_type=jnp.float32,
                         precision=HI) + bv_ref[...]
    s_ref[...] = jnp.dot(hh, ws_ref[...], preferred_element_type=jnp.float32,
                         precision=HI) + bs_ref[...]


def _proj(h, wq, wk, wv, ws, bq, bk, bv, bs, we):
    w_spec = pl.BlockSpec((HID, HID), lambda i: (0, 0))
    b_spec = pl.BlockSpec((1, HID), lambda i: (0, 0))
    r_spec = pl.BlockSpec((BNP, HID), lambda i: (i, 0))
    return pl.pallas_call(
        _proj_body,
        grid=(N // BNP,),
        in_specs=[r_spec, w_spec, w_spec, w_spec, w_spec, b_spec, b_spec,
                  b_spec, b_spec, pl.BlockSpec((EDIM, HID), lambda i: (0, 0))],
        out_specs=[pl.BlockSpec((BNP, 2 * HID), lambda i: (i, 0)),
                   r_spec, r_spec, r_spec],
        out_shape=[
            jax.ShapeDtypeStruct((N, 2 * HID), jnp.float32),
            jax.ShapeDtypeStruct((N, HID), jnp.float32),
            jax.ShapeDtypeStruct((N, HID), jnp.float32),
            jax.ShapeDtypeStruct((N, HID), jnp.float32),
        ],
    )(h, wq, wk, wv, ws, bq.reshape(1, -1), bk.reshape(1, -1),
      bv.reshape(1, -1), bs.reshape(1, -1), we)


def _redmax_body(x_ref, o_ref):
    m = jnp.max(x_ref[...], axis=0, keepdims=True)
    o_ref[...] = jnp.where(jnp.isfinite(m), m, 0.0)


def _redsum_body(x_ref, o_ref):
    o_ref[...] = jnp.sum(x_ref[...], axis=0, keepdims=True)


def _reduce_parts(parts, kind):
    body = _redmax_body if kind == "max" else _redsum_body
    out = pl.pallas_call(
        body,
        in_specs=[pl.BlockSpec((NW, N), lambda: (0, 0))],
        out_specs=pl.BlockSpec((1, N), lambda: (0, 0)),
        out_shape=jax.ShapeDtypeStruct((1, N), jnp.float32),
    )(parts)
    return out.reshape(N)


def _finish_body(ov_ref, oe_ref, s_ref, we_ref, o_ref):
    av = ov_ref[0] + ov_ref[1]
    ae = oe_ref[0] + oe_ref[1]
    h = av + jnp.dot(ae, we_ref[...], preferred_element_type=jnp.float32,
                     precision=HI) + s_ref[...]
    o_ref[...] = jnp.maximum(h, 0.0)


def _finish(outv, oute, s, we):
    return pl.pallas_call(
        _finish_body,
        grid=(N // BNP,),
        in_specs=[
            pl.BlockSpec((2, BNP, HID), lambda i: (0, i, 0)),
            pl.BlockSpec((2, BNP, EDIM), lambda i: (0, i, 0)),
            pl.BlockSpec((BNP, HID), lambda i: (i, 0)),
            pl.BlockSpec((EDIM, HID), lambda i: (0, 0)),
        ],
        out_specs=pl.BlockSpec((BNP, HID), lambda i: (i, 0)),
        out_shape=jax.ShapeDtypeStruct((N, HID), jnp.float32),
    )(outv, oute, s, we)


def _perm(x, idx):
    return lax.gather(
        x, idx[:, None],
        lax.GatherDimensionNumbers(offset_dims=(), collapsed_slice_dims=(0,),
                                   start_index_map=(0,)),
        (1,), mode=lax.GatherScatterMode.PROMISE_IN_BOUNDS)


def _seg_ends(key, iota):
    nxt = _perm(key, jnp.minimum(iota + 1, 15))
    return (key != nxt) | (iota == 15)


def _wid():
    return lax.axis_index("c") * 16 + lax.axis_index("s")


_MESH = plsc.VectorSubcoreMesh(core_axis_name="c", subcore_axis_name="s")
_SC_PARAMS = pltpu.CompilerParams(needs_layout_passes=False)


def _alpha_kernel(qcat_hbm, k_hbm, attr_hbm, src2_hbm, dst2_hbm,
                  alpha_hbm, mpart_hbm,
                  didx_v, sidx_v, qbuf, kbuf, attrbuf, alpha_v, m_v,
                  sem):
    w = _wid()
    base = w * EPW
    iota = lax.iota(jnp.int32, 16)
    neg = jnp.full((16,), -jnp.inf, jnp.float32)

    def init_m(i, _):
        m_v[pl.ds(i * 16, 16)] = neg
        return 0
    lax.fori_loop(0, N // 16, init_m, 0)

    def chunk(c, _):
        pltpu.sync_copy(dst2_hbm.at[w * NCHUNK + c], didx_v)
        pltpu.sync_copy(src2_hbm.at[w * NCHUNK + c], sidx_v)
        cp1 = pltpu.async_copy(qcat_hbm.at[didx_v], qbuf, sem)
        cp3 = pltpu.async_copy(k_hbm.at[sidx_v], kbuf, sem)
        cp4 = pltpu.async_copy(attr_hbm.at[pl.ds(base + c * C, C)], attrbuf,
                               sem)
        cp1.wait(); cp3.wait(); cp4.wait()

        def group(g, _):
            av = jnp.zeros((16,), jnp.float32)
            for j in range(16):
                e = g * 16 + j
                acc = qbuf[e, pl.ds(HID, 16)] * attrbuf[e, :]
                for r in range(HID // 16):
                    acc = acc + qbuf[e, pl.ds(r * 16, 16)] * \
                        kbuf[e, pl.ds(r * 16, 16)]
                for sh in (8, 4, 2, 1):
                    acc = acc + _perm(acc, (iota + sh) & 15)
                av = jnp.where(iota == j, acc * INV_SQRT, av)
            alpha_v[pl.ds(c * C + g * 16, 16)] = av
            dv = didx_v[pl.ds(g * 16, 16)]
            key, val = plsc.sort_key_val(dv, av)
            for sh in (1, 2, 4, 8):
                pidx = jnp.maximum(iota - sh, 0)
                pk = _perm(key, pidx)
                pv = _perm(val, pidx)
                val = jnp.where(pk == key, jnp.maximum(val, pv), val)
            last = _seg_ends(key, iota)
            old = plsc.load_gather(m_v, [key])
            plsc.store_scatter(m_v, [key], jnp.maximum(old, val), mask=last)
            return 0
        lax.fori_loop(0, NGRP, group, 0)
        return 0
    lax.fori_loop(0, NCHUNK, chunk, 0)

    pltpu.sync_copy(alpha_v, alpha_hbm.at[pl.ds(base, EPW)])
    pltpu.sync_copy(m_v, mpart_hbm.at[w])


def _sc_alpha(qcat, k, attr, src2, dst2):
    kf = pl.kernel(
        _alpha_kernel,
        mesh=_MESH,
        compiler_params=_SC_PARAMS,
        out_type=[
            jax.ShapeDtypeStruct((E,), jnp.float32),
            jax.ShapeDtypeStruct((NW, N), jnp.float32),
        ],
        scratch_types=[
            pltpu.VMEM((C,), jnp.int32),
            pltpu.VMEM((C,), jnp.int32),
            pltpu.VMEM((C, 2 * HID), jnp.float32),
            pltpu.VMEM((C, HID), jnp.float32),
            pltpu.VMEM((C, EDIM), jnp.float32),
            pltpu.VMEM((EPW,), jnp.float32),
            pltpu.VMEM((N,), jnp.float32),
            pltpu.SemaphoreType.DMA,
        ],
    )
    return kf(qcat, k, attr, src2, dst2)


def _den_kernel(alpha_hbm, m_hbm, dst_hbm, ex_hbm, denpart_hbm,
                a_v, m_v, dst_v, den_v, sem):
    w = _wid()
    base = w * EPW
    iota = lax.iota(jnp.int32, 16)
    zero = jnp.zeros((16,), jnp.float32)

    pltpu.sync_copy(alpha_hbm.at[pl.ds(base, EPW)], a_v)
    pltpu.sync_copy(dst_hbm.at[pl.ds(base, EPW)], dst_v)
    pltpu.sync_copy(m_hbm, m_v)

    def init_d(i, _):
        den_v[pl.ds(i * 16, 16)] = zero
        return 0
    lax.fori_loop(0, N // 16, init_d, 0)

    def group(g, _):
        dv = dst_v[pl.ds(g * 16, 16)]
        al = a_v[pl.ds(g * 16, 16)]
        mv = plsc.load_gather(m_v, [dv])
        ex = jnp.exp(al - mv)
        a_v[pl.ds(g * 16, 16)] = ex
        key, val = plsc.sort_key_val(dv, ex)
        for sh in (1, 2, 4, 8):
            pidx = jnp.maximum(iota - sh, 0)
            pk = _perm(key, pidx)
            pv = _perm(val, pidx)
            val = val + jnp.where((pk == key) & (iota >= sh), pv, 0.0)
        last = _seg_ends(key, iota)
        old = plsc.load_gather(den_v, [key])
        plsc.store_scatter(den_v, [key], old + val, mask=last)
        return 0
    lax.fori_loop(0, EPW // 16, group, 0)

    pltpu.sync_copy(a_v, ex_hbm.at[pl.ds(base, EPW)])
    pltpu.sync_copy(den_v, denpart_hbm.at[w])


def _sc_den(alpha, m, dst):
    kf = pl.kernel(
        _den_kernel,
        mesh=_MESH,
        compiler_params=_SC_PARAMS,
        out_type=[
            jax.ShapeDtypeStruct((E,), jnp.float32),
            jax.ShapeDtypeStruct((NW, N), jnp.float32),
        ],
        scratch_types=[
            pltpu.VMEM((EPW,), jnp.float32),
            pltpu.VMEM((N,), jnp.float32),
            pltpu.VMEM((EPW,), jnp.int32),
            pltpu.VMEM((N,), jnp.float32),
            pltpu.SemaphoreType.DMA,
        ],
    )
    return kf(alpha, m, dst)


def _copy_rows(copy_fn, nrows):
    k, rem = divmod(nrows, C)
    for i in range(k):
        copy_fn(i * C, C)
    if rem:
        copy_fn(k * C, rem)


def _out_kernel(v_hbm, attr_hbm, ex_hbm, den_hbm, src2_hbm, dst2_hbm,
                outv_hbm, oute_hbm,
                didx_v, widx_v, sidx_v, vbuf, attrbuf, attrbuf2, exbuf, den_v,
                accv, acce, sem):
    cid = lax.axis_index("c")
    sid = lax.axis_index("s")
    w = cid * 16 + sid
    base = w * EPW

    pltpu.sync_copy(den_hbm, den_v)

    def zset(i, _):
        vbuf[i // 8, pl.ds((i % 8) * 16, 16)] = jnp.zeros((16,), jnp.float32)
        return 0
    lax.fori_loop(0, C * 8, zset, 0)

    def zsete(i, _):
        attrbuf[i, :] = jnp.zeros((16,), jnp.float32)
        return 0
    lax.fori_loop(0, C, zsete, 0)

    for t in range(16):
        r0 = t * RPT
        nr = RPT if t < 15 else RPT_LAST
        e0 = t * 80
        ne = 80 if t < 15 else FEROWS - 15 * 80

        @pl.when(sid == t)
        def _(r0=r0, nr=nr, e0=e0, ne=ne):
            def zcp(off, n):
                pltpu.sync_copy(vbuf.at[pl.ds(0, n)],
                                accv.at[pl.ds(r0 + off, n)])
            _copy_rows(zcp, nr)

            def zcpe(off, n):
                pltpu.sync_copy(attrbuf2.at[pl.ds(0, n)],
                                acce.at[pl.ds(e0 + off, n)])
            _copy_rows(zcpe, ne)
    plsc.subcore_barrier()

    def chunk(c, _):
        pltpu.sync_copy(dst2_hbm.at[w * NCHUNK + c], didx_v.at[0])
        pltpu.sync_copy(src2_hbm.at[w * NCHUNK + c], sidx_v)
        cp1 = pltpu.async_copy(v_hbm.at[sidx_v], vbuf, sem)
        cp2 = pltpu.async_copy(attr_hbm.at[pl.ds(base + c * C, C)], attrbuf,
                               sem)
        cp3 = pltpu.async_copy(ex_hbm.at[pl.ds(base + c * C, C)], exbuf, sem)
        cp1.wait(); cp2.wait(); cp3.wait()

        def group(g, _):
            dv = didx_v[0, pl.ds(g * 16, 16)]
            ex = exbuf[pl.ds(g * 16, 16)]
            dn = plsc.load_gather(den_v, [dv])
            a = ex / (dn + 1e-16)
            widx_v[0, pl.ds(g * 16, 16)] = lax.shift_right_logical(dv, 3)
            w8v = jnp.bitwise_and(dv, 7)
            z16 = jnp.zeros((16,), jnp.float32)
            for j in range(16):
                e = g * 16 + j
                aj = a[j]
                for r in range(HID // 16):
                    vbuf[e, pl.ds(r * 16, 16)] = aj * vbuf[e, pl.ds(r * 16, 16)]
                av = aj * attrbuf[e, :]
                w8 = w8v[j]
                for r in range(8):
                    attrbuf2[e, pl.ds(r * 16, 16)] = jnp.where(w8 == r, av,
                                                               z16)
            return 0
        lax.fori_loop(0, NGRP, group, 0)
        pltpu.sync_copy(vbuf, accv.at[didx_v.at[0]], add=True)
        pltpu.sync_copy(attrbuf2, acce.at[widx_v.at[0]], add=True)
        return 0
    lax.fori_loop(0, NCHUNK, chunk, 0)

    plsc.subcore_barrier()

    for t in range(16):
        r0 = t * RPT
        nr = RPT if t < 15 else RPT_LAST
        e0 = t * 80
        ne = 80 if t < 15 else FEROWS - 15 * 80

        @pl.when(sid == t)
        def _(r0=r0, nr=nr, e0=e0, ne=ne):
            def ocp(off, n):
                ro = pl.multiple_of(cid * N + r0 + off, 8)
                pltpu.sync_copy(accv.at[pl.ds(r0 + off, n)],
                                vbuf.at[pl.ds(0, n)])
                pltpu.sync_copy(vbuf.at[pl.ds(0, n)],
                                outv_hbm.at[pl.ds(ro, n)])
            _copy_rows(ocp, nr)

            def ocpe(off, n):
                ro = pl.multiple_of(cid * FEROWS + e0 + off, 8)
                pltpu.sync_copy(acce.at[pl.ds(e0 + off, n)],
                                attrbuf2.at[pl.ds(0, n)])
                pltpu.sync_copy(attrbuf2.at[pl.ds(0, n)],
                                oute_hbm.at[pl.ds(ro, n)])
            _copy_rows(ocpe, ne)


def _sc_out(v, attr, ex, den, src2, dst2):
    kf = pl.kernel(
        _out_kernel,
        mesh=_MESH,
        compiler_params=_SC_PARAMS,
        out_type=[
            jax.ShapeDtypeStruct((2 * N, HID), jnp.float32),
            jax.ShapeDtypeStruct((2 * FEROWS, HID), jnp.float32),
        ],
        scratch_types=[
            pltpu.VMEM((2, C), jnp.int32),
            pltpu.VMEM((2, C), jnp.int32),
            pltpu.VMEM((C,), jnp.int32),
            pltpu.VMEM((C, HID), jnp.float32),
            pltpu.VMEM((C, EDIM), jnp.float32),
            pltpu.VMEM((C, HID), jnp.float32),
            pltpu.VMEM((C,), jnp.float32),
            pltpu.VMEM((N,), jnp.float32),
            pltpu.VMEM_SHARED((N, HID), jnp.float32),
            pltpu.VMEM_SHARED((FEROWS, HID), jnp.float32),
            pltpu.SemaphoreType.DMA,
        ],
    )
    outv, oute = kf(v, attr, ex, den, src2, dst2)
    oute = oute.reshape(2, FEROWS, HID)[:, :N * EDIM // HID].reshape(
        2, N, EDIM)
    return outv.reshape(2, N, HID), oute


def kernel(x, edge_index, edge_attr, adj, W_rel_e, W_root_e, b_root_e, ln_g,
           ln_b, Wq, bq, Wk, bk, Wv, bv, We, Wskip, bskip, W_rel_d, W_root_d,
           b_root_d):
    src = edge_index[0]
    dst = edge_index[1]
    src2 = src.reshape(E // C, C)
    dst2 = dst.reshape(E // C, C)

    xf = x.reshape(N, -1)
    adj2 = adj.reshape(N, N)
    h = _dense_sage(adj2, xf, W_rel_e, W_root_e, b_root_e, ln_g, ln_b, True)
    for i in range(N_LAYERS):
        qcat, k, v, s = _proj(h, Wq[i], Wk[i], Wv[i], Wskip[i], bq[i], bk[i],
                              bv[i], bskip[i], We[i])
        alpha, mpart = _sc_alpha(qcat, k, edge_attr, src2, dst2)
        m = _reduce_parts(mpart, "max")
        ex, denpart = _sc_den(alpha, m, dst)
        den = _reduce_parts(denpart, "sum")
        a = ex / (den[dst] + 1e-16)
        vj = v[src] + edge_attr @ We[i]
        outv = jax.ops.segment_sum(a[:, None] * vj, dst, num_segments=N)
        h = jax.nn.relu(outv + s)
    out = _dense_sage(adj2, h, W_rel_d, W_root_d, b_root_d, ln_g, ln_b, False)
    return out.reshape(N, HID, 1)
